# Initial kernel scaffold; baseline (speedup 1.0000x reference)
import jax
import jax.numpy as jnp
from jax.experimental import pallas as pl


def kernel(waveform):
    raise NotImplementedError("write your pallas kernel here")


# speedup vs baseline: 2.4576x; 2.4576x over previous
import numpy as np
import jax
import jax.numpy as jnp
from jax.experimental import pallas as pl

_SR = 44100
_SIL_LO = int(0.01 * _SR)
_SIL_HI = int(0.1 * _SR)
_RATIO_LO, _RATIO_HI = 0.1, 0.2
_LENGTH = 441000


def _build_mask(length: int) -> np.ndarray:
    rng = np.random.default_rng(0)
    cur = 0
    total_target = int(rng.integers(int(_RATIO_LO * length), int(_RATIO_HI * length)))
    mask = np.ones((length,), np.float32)
    while cur < total_target:
        sl = int(rng.integers(_SIL_LO, _SIL_HI))
        start = int(rng.integers(0, length - sl))
        mode = int(rng.integers(0, 2))
        if mode == 0:
            mask[start:start + sl] = 0.0
        else:
            fade = np.concatenate((
                np.linspace(0.0, 1.0, sl // 2, dtype=np.float32),
                np.linspace(1.0, 0.0, sl - sl // 2, dtype=np.float32),
            ))
            mask[start:start + sl] *= fade
        cur += sl
    return mask


_MASK = _build_mask(_LENGTH)

_CHUNK = 16384


def _mul_kernel(w_ref, m_ref, o_ref):
    o_ref[...] = w_ref[...] * m_ref[...]


def kernel(waveform):
    b, c, length = waveform.shape
    w2 = waveform.reshape(b * c, length)
    mask = jnp.asarray(_MASK).reshape(1, length)
    grid = (pl.cdiv(length, _CHUNK),)
    out = pl.pallas_call(
        _mul_kernel,
        grid=grid,
        in_specs=[
            pl.BlockSpec((b * c, _CHUNK), lambda i: (0, i)),
            pl.BlockSpec((1, _CHUNK), lambda i: (0, i)),
        ],
        out_specs=pl.BlockSpec((b * c, _CHUNK), lambda i: (0, i)),
        out_shape=jax.ShapeDtypeStruct((b * c, length), jnp.float32),
    )(w2, mask)
    return out.reshape(b, c, length)


# speedup vs baseline: 2.6996x; 1.0984x over previous
import numpy as np
import jax
import jax.numpy as jnp
from jax.experimental import pallas as pl
from jax.experimental.pallas import tpu as pltpu

_SR = 44100
_SIL_LO = int(0.01 * _SR)
_SIL_HI = int(0.1 * _SR)
_RATIO_LO, _RATIO_HI = 0.1, 0.2
_LENGTH = 441000


def _build_mask(length: int) -> np.ndarray:
    rng = np.random.default_rng(0)
    cur = 0
    total_target = int(rng.integers(int(_RATIO_LO * length), int(_RATIO_HI * length)))
    mask = np.ones((length,), np.float32)
    while cur < total_target:
        sl = int(rng.integers(_SIL_LO, _SIL_HI))
        start = int(rng.integers(0, length - sl))
        mode = int(rng.integers(0, 2))
        if mode == 0:
            mask[start:start + sl] = 0.0
        else:
            fade = np.concatenate((
                np.linspace(0.0, 1.0, sl // 2, dtype=np.float32),
                np.linspace(1.0, 0.0, sl - sl // 2, dtype=np.float32),
            ))
            mask[start:start + sl] *= fade
        cur += sl
    return mask


_MASK = _build_mask(_LENGTH)

_CHUNK = 49152


def _mul_kernel(w_ref, m_ref, o_ref):
    o_ref[...] = w_ref[...] * m_ref[...]


def kernel(waveform):
    b, c, length = waveform.shape
    w2 = waveform.reshape(b * c, length)
    mask = jnp.asarray(_MASK).reshape(1, length)
    grid = (pl.cdiv(length, _CHUNK),)
    out = pl.pallas_call(
        _mul_kernel,
        grid=grid,
        in_specs=[
            pl.BlockSpec((b * c, _CHUNK), lambda i: (0, i)),
            pl.BlockSpec((1, _CHUNK), lambda i: (0, i)),
        ],
        out_specs=pl.BlockSpec((b * c, _CHUNK), lambda i: (0, i)),
        out_shape=jax.ShapeDtypeStruct((b * c, length), jnp.float32),
        compiler_params=pltpu.CompilerParams(
            dimension_semantics=("parallel",),
        ),
    )(w2, mask)
    return out.reshape(b, c, length)


# speedup vs baseline: 18.1110x; 6.7089x over previous
import numpy as np
import jax
import jax.numpy as jnp
from jax.experimental import pallas as pl
from jax.experimental.pallas import tpu as pltpu

_SR = 44100
_SIL_LO = int(0.01 * _SR)
_SIL_HI = int(0.1 * _SR)
_RATIO_LO, _RATIO_HI = 0.1, 0.2
_LENGTH = 441000


def _build_mask(length: int) -> np.ndarray:
    rng = np.random.default_rng(0)
    cur = 0
    total_target = int(rng.integers(int(_RATIO_LO * length), int(_RATIO_HI * length)))
    mask = np.ones((length,), np.float32)
    while cur < total_target:
        sl = int(rng.integers(_SIL_LO, _SIL_HI))
        start = int(rng.integers(0, length - sl))
        mode = int(rng.integers(0, 2))
        if mode == 0:
            mask[start:start + sl] = 0.0
        else:
            fade = np.concatenate((
                np.linspace(0.0, 1.0, sl // 2, dtype=np.float32),
                np.linspace(1.0, 0.0, sl - sl // 2, dtype=np.float32),
            ))
            mask[start:start + sl] *= fade
        cur += sl
    return mask


_MASK = _build_mask(_LENGTH)

_CHUNK = 49152


def _mul_kernel(w_ref, m_ref, o_ref):
    o_ref[...] = w_ref[...] * m_ref[...]


def kernel(waveform):
    b, c, length = waveform.shape
    mask = jnp.asarray(_MASK).reshape(1, 1, length)
    grid = (pl.cdiv(length, _CHUNK),)
    out = pl.pallas_call(
        _mul_kernel,
        grid=grid,
        in_specs=[
            pl.BlockSpec((b, c, _CHUNK), lambda i: (0, 0, i)),
            pl.BlockSpec((1, c, _CHUNK), lambda i: (0, 0, i)),
        ],
        out_specs=pl.BlockSpec((b, c, _CHUNK), lambda i: (0, 0, i)),
        out_shape=jax.ShapeDtypeStruct((b, c, length), jnp.float32),
        compiler_params=pltpu.CompilerParams(
            dimension_semantics=("parallel",),
        ),
    )(waveform, mask)
    return out
